# Initial kernel scaffold; baseline (speedup 1.0000x reference)
import jax
import jax.numpy as jnp
from jax.experimental import pallas as pl


def kernel(x, edge_index, batch, W1, b1, W2, b2, Wc1, bc1, Wc2, bc2, Wc3, bc3, Wg1, bg1, bn_gamma, bn_beta, bn_mean, bn_var, Wg2, bg2):
    raise NotImplementedError("write your pallas kernel here")


# speedup vs baseline: 1.0002x; 1.0002x over previous
import jax
import jax.numpy as jnp
from jax.experimental import pallas as pl


def _fused_feat_conv1_kernel(xa_ref, xb_ref, xc_ref, W2_ref, b2_ref,
                             W1_ref, b1_ref, Wa_ref, Wb_ref, Wc_ref, o_ref):
    f2 = jax.nn.relu(xa_ref[...] @ W2_ref[...] + b2_ref[...])
    f1 = jax.nn.relu(xb_ref[...] @ W1_ref[...] + b1_ref[...])
    o_ref[...] = f2 @ Wa_ref[...] + f1 @ Wb_ref[...] + xc_ref[...] @ Wc_ref[...]


def _bias_relu_mm_kernel(a_ref, b_ref, W_ref, o_ref):
    h = jax.nn.relu(a_ref[...] + b_ref[...])
    o_ref[...] = h @ W_ref[...]


def _pool_kernel(P_ref, a_ref, b_ref, o_ref):
    k = pl.program_id(0)
    h = jax.nn.relu(a_ref[...] + b_ref[...])
    part = P_ref[...] @ h

    @pl.when(k == 0)
    def _():
        o_ref[...] = part

    @pl.when(k > 0)
    def _():
        o_ref[...] += part


def _head_kernel(ps_ref, ci_ref, Wg1_ref, bg1_ref, gm_ref, gv_ref,
                 gg_ref, gb_ref, Wg2_ref, bg2_ref, o_ref):
    pooled = ps_ref[...] * ci_ref[...]
    g = pooled @ Wg1_ref[...] + bg1_ref[...]
    g = (g - gm_ref[...]) / jnp.sqrt(gv_ref[...] + 1e-5) * gg_ref[...] + gb_ref[...]
    g = jax.nn.relu(g)
    o_ref[...] = jax.nn.sigmoid(g @ Wg2_ref[...] + bg2_ref[...])


def _full_spec(shape):
    nd = len(shape)
    return pl.BlockSpec(shape, lambda i, _nd=nd: (0,) * _nd)


def kernel(x, edge_index, batch, W1, b1, W2, b2, Wc1, bc1, Wc2, bc2, Wc3, bc3,
           Wg1, bg1, bn_gamma, bn_beta, bn_mean, bn_var, Wg2, bg2):
    N = x.shape[0]
    B = 64
    f32 = jnp.float32

    src = edge_index[0]
    dst = edge_index[1]
    loop = jnp.arange(N, dtype=src.dtype)
    src2 = jnp.concatenate([src, loop])
    dst2 = jnp.concatenate([dst, loop])
    deg = jax.ops.segment_sum(jnp.ones_like(dst2, dtype=f32), dst2, num_segments=N)
    dinv = jnp.where(deg > 0, 1.0 / jnp.sqrt(deg), 0.0)
    norm = (dinv[src2] * dinv[dst2])[:, None]

    def agg(m):
        return jax.ops.segment_sum(m[src2] * norm, dst2, num_segments=N)

    xa = x[:, :21]
    xb = x[:, 21:6165]
    xc = x[:, 6165:]
    Wa, Wb, Wc = Wc1[:21], Wc1[21:149], Wc1[149:]
    D1 = Wc1.shape[1]

    BM = 400
    g1 = pl.cdiv(N, BM)
    m1 = pl.pallas_call(
        _fused_feat_conv1_kernel,
        grid=(g1,),
        in_specs=[
            pl.BlockSpec((BM, 21), lambda i: (i, 0)),
            pl.BlockSpec((BM, 6144), lambda i: (i, 0)),
            pl.BlockSpec((BM, 320), lambda i: (i, 0)),
            _full_spec((21, 21)),
            _full_spec((1, 21)),
            _full_spec((6144, 128)),
            _full_spec((1, 128)),
            _full_spec((21, D1)),
            _full_spec((128, D1)),
            _full_spec((320, D1)),
        ],
        out_specs=pl.BlockSpec((BM, D1), lambda i: (i, 0)),
        out_shape=jax.ShapeDtypeStruct((N, D1), f32),
    )(xa, xb, xc, W2, b2[None, :], W1, b1[None, :], Wa, Wb, Wc)

    a1 = agg(m1)

    def bias_relu_mm(a, b, W, BM=512):
        n, K = a.shape
        Nn = W.shape[1]
        return pl.pallas_call(
            _bias_relu_mm_kernel,
            grid=(pl.cdiv(n, BM),),
            in_specs=[
                pl.BlockSpec((BM, K), lambda i: (i, 0)),
                _full_spec((1, K)),
                _full_spec((K, Nn)),
            ],
            out_specs=pl.BlockSpec((BM, Nn), lambda i: (i, 0)),
            out_shape=jax.ShapeDtypeStruct((n, Nn), f32),
        )(a, b[None, :], W)

    m2 = bias_relu_mm(a1, bc1, Wc2)
    a2 = agg(m2)
    m3 = bias_relu_mm(a2, bc2, Wc3)
    a3 = agg(m3)

    D4 = Wc3.shape[1]
    P = (batch[None, :] == jnp.arange(B, dtype=batch.dtype)[:, None]).astype(f32)
    cnt = jnp.sum(P, axis=1)
    cntinv = (1.0 / jnp.maximum(cnt, 1.0))[:, None]

    BMP = 1024
    Np = ((N + BMP - 1) // BMP) * BMP
    P_pad = jnp.pad(P, ((0, 0), (0, Np - N)))
    a3_pad = jnp.pad(a3, ((0, Np - N), (0, 0)))
    pooled_sum = pl.pallas_call(
        _pool_kernel,
        grid=(Np // BMP,),
        in_specs=[
            pl.BlockSpec((B, BMP), lambda i: (0, i)),
            pl.BlockSpec((BMP, D4), lambda i: (i, 0)),
            _full_spec((1, D4)),
        ],
        out_specs=pl.BlockSpec((B, D4), lambda i: (0, 0)),
        out_shape=jax.ShapeDtypeStruct((B, D4), f32),
    )(P_pad, a3_pad, bc3[None, :])

    out = pl.pallas_call(
        _head_kernel,
        out_shape=jax.ShapeDtypeStruct((B, Wg2.shape[1]), f32),
    )(pooled_sum, cntinv, Wg1, bg1[None, :], bn_mean[None, :],
      bn_var[None, :], bn_gamma[None, :], bn_beta[None, :], Wg2, bg2[None, :])
    return out


# speedup vs baseline: 2.4561x; 2.4558x over previous
import functools

import jax
import jax.numpy as jnp
from jax import lax
from jax.experimental import pallas as pl
from jax.experimental.pallas import tpu as pltpu
from jax.experimental.pallas import tpu_sc as plsc

_N = 10000
_E2 = 170000
_WT = 10752
_E2P = 16 * _WT
_NACC = 10112
_RPT = _NACC // 16


def _fused_feat_conv1_kernel(xa_ref, xb_ref, xc_ref, W2_ref, b2_ref,
                             W1_ref, b1_ref, Wa_ref, Wb_ref, Wc_ref,
                             dinv_ref, o_ref):
    f2 = jax.nn.relu(xa_ref[...] @ W2_ref[...] + b2_ref[...])
    f1 = jax.nn.relu(xb_ref[...] @ W1_ref[...] + b1_ref[...])
    acc = f2 @ Wa_ref[...] + f1 @ Wb_ref[...] + xc_ref[...] @ Wc_ref[...]
    o_ref[...] = acc * dinv_ref[...]


def _bias_relu_mm_kernel(a_ref, b_ref, W_ref, dinv_ref, o_ref):
    h = jax.nn.relu(a_ref[...] * dinv_ref[...] + b_ref[...])
    o_ref[...] = (h @ W_ref[...]) * dinv_ref[...]


def _pool_kernel(P_ref, a_ref, b_ref, dinv_ref, o_ref):
    k = pl.program_id(0)
    h = jax.nn.relu(a_ref[...] * dinv_ref[...] + b_ref[...])
    part = P_ref[...] @ h

    @pl.when(k == 0)
    def _():
        o_ref[...] = part

    @pl.when(k > 0)
    def _():
        o_ref[...] += part


def _head_kernel(ps_ref, ci_ref, Wg1_ref, bg1_ref, gm_ref, gv_ref,
                 gg_ref, gb_ref, Wg2_ref, bg2_ref, o_ref):
    pooled = ps_ref[...] * ci_ref[...]
    g = pooled @ Wg1_ref[...] + bg1_ref[...]
    g = (g - gm_ref[...]) / jnp.sqrt(gv_ref[...] + 1e-5) * gg_ref[...] + gb_ref[...]
    g = jax.nn.relu(g)
    o_ref[...] = jax.nn.sigmoid(g @ Wg2_ref[...] + bg2_ref[...])


def _full_spec(shape):
    nd = len(shape)
    return pl.BlockSpec(shape, lambda i, _nd=nd: (0,) * _nd)


_sc_mesh = plsc.VectorSubcoreMesh(core_axis_name="c", subcore_axis_name="s")


@functools.partial(
    pl.kernel,
    mesh=_sc_mesh,
    out_type=[
        jax.ShapeDtypeStruct((_NACC, 128), jnp.float32),
        jax.ShapeDtypeStruct((_NACC, 128), jnp.float32),
    ],
    scratch_types=[
        pltpu.VMEM_SHARED((_NACC, 128), jnp.float32),
        pltpu.VMEM((128,), jnp.int32),
        pltpu.VMEM((128,), jnp.int32),
        pltpu.VMEM((128, 128), jnp.float32),
        pltpu.SemaphoreType.DMA,
    ],
)
def _sc_agg_pair(t0, t1, src_ref, dst_ref, zr_ref, o0, o1,
                 acc, idxv, dstv, rows, sem):
    c = lax.axis_index("c")
    s = lax.axis_index("s")
    pltpu.sync_copy(zr_ref, acc.at[pl.ds(s * _RPT, _RPT)])
    plsc.subcore_barrier()

    def run(table, out):
        def step(j, carry):
            base = s * _WT + j * 128
            pltpu.sync_copy(src_ref.at[pl.ds(base, 128)], idxv)
            pltpu.async_copy(table.at[idxv], rows, sem).wait()
            pltpu.sync_copy(dst_ref.at[pl.ds(base, 128)], dstv)
            pltpu.sync_copy(rows, acc.at[dstv], add=True)
            return carry

        lax.fori_loop(0, _WT // 128, step, 0)
        plsc.subcore_barrier()
        pltpu.sync_copy(acc.at[pl.ds(s * _RPT, _RPT)],
                        out.at[pl.ds(s * _RPT, _RPT)])

    @pl.when(c == 0)
    def _():
        run(t0, o0)

    @pl.when(c == 1)
    def _():
        run(t1, o1)


def kernel(x, edge_index, batch, W1, b1, W2, b2, Wc1, bc1, Wc2, bc2, Wc3, bc3,
           Wg1, bg1, bn_gamma, bn_beta, bn_mean, bn_var, Wg2, bg2):
    N = _N
    B = 64
    f32 = jnp.float32

    src = edge_index[0]
    dst = edge_index[1]
    loop = jnp.arange(N, dtype=src.dtype)
    src2 = jnp.concatenate([src, loop])
    dst2 = jnp.concatenate([dst, loop])
    deg = jax.ops.segment_sum(jnp.ones_like(dst2, dtype=f32), dst2, num_segments=N)
    dinv = jnp.where(deg > 0, 1.0 / jnp.sqrt(deg), 0.0)[:, None]

    npad = _E2P - _E2
    srcp = jnp.concatenate([src2, jnp.zeros((npad,), src.dtype)])
    dstp = jnp.concatenate([dst2, jnp.full((npad,), N, src.dtype)])
    zr = jnp.zeros((_RPT, 128), f32)
    zchunk = jnp.zeros((N, 128), f32)

    def sc_agg(m):
        C = m.shape[1] // 128
        outs = []
        for c0 in range(0, C, 2):
            t0 = m[:, c0 * 128:(c0 + 1) * 128]
            if c0 + 1 < C:
                t1 = m[:, (c0 + 1) * 128:(c0 + 2) * 128]
            else:
                t1 = zchunk
            o0, o1 = _sc_agg_pair(t0, t1, srcp, dstp, zr)
            outs.append(o0[:N])
            if c0 + 1 < C:
                outs.append(o1[:N])
        return jnp.concatenate(outs, axis=1)

    D1p, D2p, D3p = 512, 1024, 1920
    pad = lambda a, r, c: jnp.pad(a, ((0, r), (0, c)))
    Wa = pad(Wc1[:21], 0, D1p - 469)
    Wb = pad(Wc1[21:149], 0, D1p - 469)
    Wc = pad(Wc1[149:], 0, D1p - 469)
    bc1p = jnp.pad(bc1, (0, D1p - 469))
    Wc2p = pad(Wc2, D1p - 469, D2p - 938)
    bc2p = jnp.pad(bc2, (0, D2p - 938))
    Wc3p = pad(Wc3, D2p - 938, D3p - 1876)
    bc3p = jnp.pad(bc3, (0, D3p - 1876))
    Wg1p = jnp.pad(Wg1, ((0, D3p - 1876), (0, 0)))

    xa = x[:, :21]
    xb = x[:, 21:6165]
    xc = x[:, 6165:]

    BM = 400
    m1 = pl.pallas_call(
        _fused_feat_conv1_kernel,
        grid=(pl.cdiv(N, BM),),
        in_specs=[
            pl.BlockSpec((BM, 21), lambda i: (i, 0)),
            pl.BlockSpec((BM, 6144), lambda i: (i, 0)),
            pl.BlockSpec((BM, 320), lambda i: (i, 0)),
            _full_spec((21, 21)),
            _full_spec((1, 21)),
            _full_spec((6144, 128)),
            _full_spec((1, 128)),
            _full_spec((21, D1p)),
            _full_spec((128, D1p)),
            _full_spec((320, D1p)),
            pl.BlockSpec((BM, 1), lambda i: (i, 0)),
        ],
        out_specs=pl.BlockSpec((BM, D1p), lambda i: (i, 0)),
        out_shape=jax.ShapeDtypeStruct((N, D1p), f32),
    )(xa, xb, xc, W2, b2[None, :], W1, b1[None, :], Wa, Wb, Wc, dinv)

    a1 = sc_agg(m1)

    def bias_relu_mm(a, b, W, BM=512):
        n, K = a.shape
        Nn = W.shape[1]
        return pl.pallas_call(
            _bias_relu_mm_kernel,
            grid=(pl.cdiv(n, BM),),
            in_specs=[
                pl.BlockSpec((BM, K), lambda i: (i, 0)),
                _full_spec((1, K)),
                _full_spec((K, Nn)),
                pl.BlockSpec((BM, 1), lambda i: (i, 0)),
            ],
            out_specs=pl.BlockSpec((BM, Nn), lambda i: (i, 0)),
            out_shape=jax.ShapeDtypeStruct((n, Nn), f32),
        )(a, b[None, :], W, dinv)

    m2 = bias_relu_mm(a1, bc1p, Wc2p)
    a2 = sc_agg(m2)
    m3 = bias_relu_mm(a2, bc2p, Wc3p)
    a3 = sc_agg(m3)

    P = (batch[None, :] == jnp.arange(B, dtype=batch.dtype)[:, None]).astype(f32)
    cnt = jnp.sum(P, axis=1)
    cntinv = (1.0 / jnp.maximum(cnt, 1.0))[:, None]

    BMP = 1024
    Np = ((N + BMP - 1) // BMP) * BMP
    P_pad = jnp.pad(P, ((0, 0), (0, Np - N)))
    a3_pad = jnp.pad(a3, ((0, Np - N), (0, 0)))
    dinv_pad = jnp.pad(dinv, ((0, Np - N), (0, 0)))
    pooled_sum = pl.pallas_call(
        _pool_kernel,
        grid=(Np // BMP,),
        in_specs=[
            pl.BlockSpec((B, BMP), lambda i: (0, i)),
            pl.BlockSpec((BMP, D3p), lambda i: (i, 0)),
            _full_spec((1, D3p)),
            pl.BlockSpec((BMP, 1), lambda i: (i, 0)),
        ],
        out_specs=pl.BlockSpec((B, D3p), lambda i: (0, 0)),
        out_shape=jax.ShapeDtypeStruct((B, D3p), f32),
    )(P_pad, a3_pad, bc3p[None, :], dinv_pad)

    out = pl.pallas_call(
        _head_kernel,
        out_shape=jax.ShapeDtypeStruct((B, Wg2.shape[1]), f32),
    )(pooled_sum, cntinv, Wg1p, bg1[None, :], bn_mean[None, :],
      bn_var[None, :], bn_gamma[None, :], bn_beta[None, :], Wg2, bg2[None, :])
    return out


# speedup vs baseline: 2.7151x; 1.1054x over previous
import functools

import jax
import jax.numpy as jnp
from jax import lax
from jax.experimental import pallas as pl
from jax.experimental.pallas import tpu as pltpu
from jax.experimental.pallas import tpu_sc as plsc

_N = 10000
_E2 = 170000
_WT = 10752
_E2P = 16 * _WT
_NACC = 10112
_RPT = _NACC // 16


def _fused_feat_conv1_kernel(xa_ref, xb_ref, xc_ref, W2_ref, b2_ref,
                             W1_ref, b1_ref, Wa_ref, Wb_ref, Wc_ref,
                             dinv_ref, o_ref):
    f2 = jax.nn.relu(xa_ref[...] @ W2_ref[...] + b2_ref[...])
    f1 = jax.nn.relu(xb_ref[...] @ W1_ref[...] + b1_ref[...])
    acc = f2 @ Wa_ref[...] + f1 @ Wb_ref[...] + xc_ref[...] @ Wc_ref[...]
    o_ref[...] = acc * dinv_ref[...]


def _bias_relu_mm_kernel(a_ref, b_ref, W_ref, dinv_ref, o_ref):
    h = jax.nn.relu(a_ref[...] * dinv_ref[...] + b_ref[...])
    o_ref[...] = (h @ W_ref[...]) * dinv_ref[...]


def _pool_kernel(P_ref, a_ref, b_ref, dinv_ref, o_ref):
    k = pl.program_id(0)
    h = jax.nn.relu(a_ref[...] * dinv_ref[...] + b_ref[...])
    part = P_ref[...] @ h

    @pl.when(k == 0)
    def _():
        o_ref[...] = part

    @pl.when(k > 0)
    def _():
        o_ref[...] += part


def _head_kernel(ps_ref, ci_ref, Wg1_ref, bg1_ref, gm_ref, gv_ref,
                 gg_ref, gb_ref, Wg2_ref, bg2_ref, o_ref):
    pooled = ps_ref[...] * ci_ref[...]
    g = pooled @ Wg1_ref[...] + bg1_ref[...]
    g = (g - gm_ref[...]) / jnp.sqrt(gv_ref[...] + 1e-5) * gg_ref[...] + gb_ref[...]
    g = jax.nn.relu(g)
    o_ref[...] = jax.nn.sigmoid(g @ Wg2_ref[...] + bg2_ref[...])


def _full_spec(shape):
    nd = len(shape)
    return pl.BlockSpec(shape, lambda i, _nd=nd: (0,) * _nd)


_sc_mesh = plsc.VectorSubcoreMesh(core_axis_name="c", subcore_axis_name="s")


@functools.partial(
    pl.kernel,
    mesh=_sc_mesh,
    out_type=[
        jax.ShapeDtypeStruct((_NACC, 128), jnp.float32),
        jax.ShapeDtypeStruct((_NACC, 128), jnp.float32),
    ],
    scratch_types=[
        pltpu.VMEM_SHARED((_NACC, 128), jnp.float32),
        pltpu.VMEM((_WT // 64, 64), jnp.int32),
        pltpu.VMEM((64,), jnp.int32),
        pltpu.VMEM((64,), jnp.int32),
        pltpu.VMEM((64, 128), jnp.float32),
        pltpu.VMEM((64, 128), jnp.float32),
        pltpu.SemaphoreType.DMA,
        pltpu.SemaphoreType.DMA,
    ],
)
def _sc_agg_pair(t0, t1, src_ref, dst_ref, zr_ref, o0, o1,
                 acc, idx_all, dst0, dst1, rows0, rows1, sem0, sem1):
    c = lax.axis_index("c")
    s = lax.axis_index("s")
    pltpu.sync_copy(zr_ref, acc.at[pl.ds(s * _RPT, _RPT)])
    plsc.subcore_barrier()

    def run(table, out):
        pltpu.sync_copy(src_ref.at[s], idx_all)

        def step(gg, carry):
            g0 = gg * 2
            g1 = g0 + 1
            cp0 = pltpu.async_copy(table.at[idx_all.at[g0]], rows0, sem0)
            cp1 = pltpu.async_copy(table.at[idx_all.at[g1]], rows1, sem1)
            pltpu.sync_copy(dst_ref.at[pl.ds(s * _WT + g0 * 64, 64)], dst0)
            cp0.wait()
            pltpu.sync_copy(rows0, acc.at[dst0], add=True)
            pltpu.sync_copy(dst_ref.at[pl.ds(s * _WT + g1 * 64, 64)], dst1)
            cp1.wait()
            pltpu.sync_copy(rows1, acc.at[dst1], add=True)
            return carry

        lax.fori_loop(0, _WT // 128, step, 0)
        plsc.subcore_barrier()
        pltpu.sync_copy(acc.at[pl.ds(s * _RPT, _RPT)],
                        out.at[pl.ds(s * _RPT, _RPT)])

    @pl.when(c == 0)
    def _():
        run(t0, o0)

    @pl.when(c == 1)
    def _():
        run(t1, o1)


def kernel(x, edge_index, batch, W1, b1, W2, b2, Wc1, bc1, Wc2, bc2, Wc3, bc3,
           Wg1, bg1, bn_gamma, bn_beta, bn_mean, bn_var, Wg2, bg2):
    N = _N
    B = 64
    f32 = jnp.float32

    src = edge_index[0]
    dst = edge_index[1]
    loop = jnp.arange(N, dtype=src.dtype)
    src2 = jnp.concatenate([src, loop])
    dst2 = jnp.concatenate([dst, loop])
    deg = jax.ops.segment_sum(jnp.ones_like(dst2, dtype=f32), dst2, num_segments=N)
    dinv = jnp.where(deg > 0, 1.0 / jnp.sqrt(deg), 0.0)[:, None]

    npad = _E2P - _E2
    srcp = jnp.concatenate([src2, jnp.zeros((npad,), src.dtype)]
                           ).reshape(16, _WT // 64, 64)
    dstp = jnp.concatenate([dst2, jnp.full((npad,), N, src.dtype)])
    zr = jnp.zeros((_RPT, 128), f32)
    zchunk = jnp.zeros((N, 128), f32)

    def sc_agg(m):
        C = m.shape[1] // 128
        outs = []
        for c0 in range(0, C, 2):
            t0 = m[:, c0 * 128:(c0 + 1) * 128]
            if c0 + 1 < C:
                t1 = m[:, (c0 + 1) * 128:(c0 + 2) * 128]
            else:
                t1 = zchunk
            o0, o1 = _sc_agg_pair(t0, t1, srcp, dstp, zr)
            outs.append(o0[:N])
            if c0 + 1 < C:
                outs.append(o1[:N])
        return jnp.concatenate(outs, axis=1)

    D1p, D2p, D3p = 512, 1024, 1920
    pad = lambda a, r, c: jnp.pad(a, ((0, r), (0, c)))
    Wa = pad(Wc1[:21], 0, D1p - 469)
    Wb = pad(Wc1[21:149], 0, D1p - 469)
    Wc = pad(Wc1[149:], 0, D1p - 469)
    bc1p = jnp.pad(bc1, (0, D1p - 469))
    Wc2p = pad(Wc2, D1p - 469, D2p - 938)
    bc2p = jnp.pad(bc2, (0, D2p - 938))
    Wc3p = pad(Wc3, D2p - 938, D3p - 1876)
    bc3p = jnp.pad(bc3, (0, D3p - 1876))
    Wg1p = jnp.pad(Wg1, ((0, D3p - 1876), (0, 0)))

    xa = x[:, :21]
    xb = x[:, 21:6165]
    xc = x[:, 6165:]

    BM = 400
    m1 = pl.pallas_call(
        _fused_feat_conv1_kernel,
        grid=(pl.cdiv(N, BM),),
        in_specs=[
            pl.BlockSpec((BM, 21), lambda i: (i, 0)),
            pl.BlockSpec((BM, 6144), lambda i: (i, 0)),
            pl.BlockSpec((BM, 320), lambda i: (i, 0)),
            _full_spec((21, 21)),
            _full_spec((1, 21)),
            _full_spec((6144, 128)),
            _full_spec((1, 128)),
            _full_spec((21, D1p)),
            _full_spec((128, D1p)),
            _full_spec((320, D1p)),
            pl.BlockSpec((BM, 1), lambda i: (i, 0)),
        ],
        out_specs=pl.BlockSpec((BM, D1p), lambda i: (i, 0)),
        out_shape=jax.ShapeDtypeStruct((N, D1p), f32),
    )(xa, xb, xc, W2, b2[None, :], W1, b1[None, :], Wa, Wb, Wc, dinv)

    a1 = sc_agg(m1)

    def bias_relu_mm(a, b, W, BM=512):
        n, K = a.shape
        Nn = W.shape[1]
        return pl.pallas_call(
            _bias_relu_mm_kernel,
            grid=(pl.cdiv(n, BM),),
            in_specs=[
                pl.BlockSpec((BM, K), lambda i: (i, 0)),
                _full_spec((1, K)),
                _full_spec((K, Nn)),
                pl.BlockSpec((BM, 1), lambda i: (i, 0)),
            ],
            out_specs=pl.BlockSpec((BM, Nn), lambda i: (i, 0)),
            out_shape=jax.ShapeDtypeStruct((n, Nn), f32),
        )(a, b[None, :], W, dinv)

    m2 = bias_relu_mm(a1, bc1p, Wc2p)
    a2 = sc_agg(m2)
    m3 = bias_relu_mm(a2, bc2p, Wc3p)
    a3 = sc_agg(m3)

    P = (batch[None, :] == jnp.arange(B, dtype=batch.dtype)[:, None]).astype(f32)
    cnt = jnp.sum(P, axis=1)
    cntinv = (1.0 / jnp.maximum(cnt, 1.0))[:, None]

    BMP = 1024
    Np = ((N + BMP - 1) // BMP) * BMP
    P_pad = jnp.pad(P, ((0, 0), (0, Np - N)))
    a3_pad = jnp.pad(a3, ((0, Np - N), (0, 0)))
    dinv_pad = jnp.pad(dinv, ((0, Np - N), (0, 0)))
    pooled_sum = pl.pallas_call(
        _pool_kernel,
        grid=(Np // BMP,),
        in_specs=[
            pl.BlockSpec((B, BMP), lambda i: (0, i)),
            pl.BlockSpec((BMP, D3p), lambda i: (i, 0)),
            _full_spec((1, D3p)),
            pl.BlockSpec((BMP, 1), lambda i: (i, 0)),
        ],
        out_specs=pl.BlockSpec((B, D3p), lambda i: (0, 0)),
        out_shape=jax.ShapeDtypeStruct((B, D3p), f32),
    )(P_pad, a3_pad, bc3p[None, :], dinv_pad)

    out = pl.pallas_call(
        _head_kernel,
        out_shape=jax.ShapeDtypeStruct((B, Wg2.shape[1]), f32),
    )(pooled_sum, cntinv, Wg1p, bg1[None, :], bn_mean[None, :],
      bn_var[None, :], bn_gamma[None, :], bn_beta[None, :], Wg2, bg2[None, :])
    return out
